# baseline (speedup 1.0000x reference)
import functools

import jax
import jax.numpy as jnp
from jax.experimental import pallas as pl
from jax.experimental.pallas import tpu as pltpu

_S = 64.0
_M = 0.15


def _loss_kernel(target_ref, x_ref, out_ref, m_sc, s_sc, t_sc, *, c_total,
                 block_c):
    k = pl.program_id(0)
    nk = pl.num_programs(0)

    x = x_ref[...] * _S
    cols = k * block_c + jax.lax.broadcasted_iota(jnp.int32, x.shape, 1)
    x = jnp.where(cols < c_total, x, -jnp.inf)

    @pl.when(k == 0)
    def _init():
        m_sc[...] = jnp.full_like(m_sc, -jnp.inf)
        s_sc[...] = jnp.zeros_like(s_sc)
        t_sc[...] = jnp.zeros_like(t_sc)

    m_old = m_sc[...]
    m_new = jnp.maximum(m_old, jnp.max(x, axis=1, keepdims=True))
    s_blk = jnp.sum(jnp.exp(x - m_new), axis=1, keepdims=True)
    s_sc[...] = s_sc[...] * jnp.exp(m_old - m_new) + s_blk
    m_sc[...] = m_new

    is_t = cols == target_ref[...]
    t_sc[...] += jnp.sum(jnp.where(is_t, x, 0.0), axis=1, keepdims=True)

    @pl.when(k == nk - 1)
    def _fin():
        m = m_sc[...]
        tv = t_sc[...]
        tm = tv - _S * _M
        se = s_sc[...] - jnp.exp(tv - m) + jnp.exp(tm - m)
        logpt = tm - m - jnp.log(se)
        out_ref[...] = jnp.full_like(out_ref, -jnp.mean(logpt))


def kernel(cos_theta, target):
    b, c = cos_theta.shape
    block_c = min(4096, c)
    grid = pl.cdiv(c, block_c)
    target2 = target.reshape(b, 1).astype(jnp.int32)

    out = pl.pallas_call(
        functools.partial(_loss_kernel, c_total=c, block_c=block_c),
        grid=(grid,),
        in_specs=[
            pl.BlockSpec((b, 1), lambda k: (0, 0)),
            pl.BlockSpec((b, block_c), lambda k: (0, k)),
        ],
        out_specs=pl.BlockSpec((1, 1), lambda k: (0, 0)),
        out_shape=jax.ShapeDtypeStruct((1, 1), jnp.float32),
        scratch_shapes=[
            pltpu.VMEM((b, 1), jnp.float32),
            pltpu.VMEM((b, 1), jnp.float32),
            pltpu.VMEM((b, 1), jnp.float32),
        ],
        compiler_params=pltpu.CompilerParams(
            dimension_semantics=("arbitrary",),
        ),
    )(target2, cos_theta)
    return out[0, 0]


# speedup vs baseline: 1.1024x; 1.1024x over previous
import functools
import math

import jax
import jax.numpy as jnp
from jax.experimental import pallas as pl
from jax.experimental.pallas import tpu as pltpu

_S = 64.0
_M = 0.15
_A = _S * math.log2(math.e)


def _loss_kernel(target_ref, x_ref, out_ref, m_sc, s_sc, t_sc, *, c_total,
                 block_c):
    k = pl.program_id(0)
    nk = pl.num_programs(0)

    @pl.when(k == 0)
    def _init():
        m_sc[...] = jnp.full_like(m_sc, -jnp.inf)
        s_sc[...] = jnp.zeros_like(s_sc)
        t_sc[...] = jnp.zeros_like(t_sc)

    def process(x):
        m_old = m_sc[...]
        m_new = jnp.maximum(m_old, jnp.max(x, axis=1, keepdims=True))
        y = jnp.exp2(x * _A - m_new * _A)
        s_blk = jnp.sum(y, axis=1, keepdims=True)
        s_sc[...] = s_sc[...] * jnp.exp2(_A * (m_old - m_new)) + s_blk
        m_sc[...] = m_new
        local_t = target_ref[...] - k * block_c
        is_t = jax.lax.broadcasted_iota(jnp.int32, x.shape, 1) == local_t
        t_sc[...] += jnp.sum(jnp.where(is_t, x, 0.0), axis=1, keepdims=True)

    @pl.when(k < nk - 1)
    def _steady():
        process(x_ref[...])

    @pl.when(k == nk - 1)
    def _last():
        valid = c_total - (nk - 1) * block_c
        x = x_ref[...]
        pad = jax.lax.broadcasted_iota(jnp.int32, x.shape, 1) >= valid
        process(jnp.where(pad, -jnp.inf, x))

        m = _S * m_sc[...]
        tv = _S * t_sc[...]
        tm = tv - _S * _M
        se = s_sc[...] - jnp.exp(tv - m) + jnp.exp(tm - m)
        logpt = tm - m - jnp.log(se)
        out_ref[...] = jnp.full_like(out_ref, -jnp.mean(logpt))


def kernel(cos_theta, target):
    b, c = cos_theta.shape
    block_c = min(4096, c)
    grid = pl.cdiv(c, block_c)
    target2 = target.reshape(b, 1).astype(jnp.int32)

    out = pl.pallas_call(
        functools.partial(_loss_kernel, c_total=c, block_c=block_c),
        grid=(grid,),
        in_specs=[
            pl.BlockSpec((b, 1), lambda k: (0, 0)),
            pl.BlockSpec((b, block_c), lambda k: (0, k)),
        ],
        out_specs=pl.BlockSpec((1, 1), lambda k: (0, 0)),
        out_shape=jax.ShapeDtypeStruct((1, 1), jnp.float32),
        scratch_shapes=[
            pltpu.VMEM((b, 1), jnp.float32),
            pltpu.VMEM((b, 1), jnp.float32),
            pltpu.VMEM((b, 1), jnp.float32),
        ],
        compiler_params=pltpu.CompilerParams(
            dimension_semantics=("arbitrary",),
        ),
    )(target2, cos_theta)
    return out[0, 0]


# speedup vs baseline: 3.3811x; 3.0670x over previous
import functools
import math

import jax
import jax.numpy as jnp
from jax.experimental import pallas as pl
from jax.experimental.pallas import tpu as pltpu

_S = 64.0
_M = 0.15
_A = _S * math.log2(math.e)


def _loss_kernel(target_ref, x_ref, out_ref, m_sc, s_sc, t_sc, *, c_total,
                 block_r):
    k = pl.program_id(0)
    nk = pl.num_programs(0)

    @pl.when(k == 0)
    def _init():
        m_sc[...] = jnp.full_like(m_sc, -jnp.inf)
        s_sc[...] = jnp.zeros_like(s_sc)
        t_sc[...] = jnp.zeros_like(t_sc)

    def process(x):
        m_old = m_sc[...]
        m_new = jnp.maximum(m_old, jnp.max(x, axis=0, keepdims=True))
        y = jnp.exp2(x * _A - m_new * _A)
        s_blk = jnp.sum(y, axis=0, keepdims=True)
        s_sc[...] = s_sc[...] * jnp.exp2(_A * (m_old - m_new)) + s_blk
        m_sc[...] = m_new
        local_t = target_ref[...] - k * block_r
        is_t = jax.lax.broadcasted_iota(jnp.int32, x.shape, 0) == local_t
        t_sc[...] += jnp.sum(jnp.where(is_t, x, 0.0), axis=0, keepdims=True)

    @pl.when(k < nk - 1)
    def _steady():
        process(x_ref[...])

    @pl.when(k == nk - 1)
    def _last():
        valid = c_total - (nk - 1) * block_r
        x = x_ref[...]
        pad = jax.lax.broadcasted_iota(jnp.int32, x.shape, 0) >= valid
        process(jnp.where(pad, -jnp.inf, x))

        m = _S * m_sc[...]
        tv = _S * t_sc[...]
        tm = tv - _S * _M
        se = s_sc[...] - jnp.exp(tv - m) + jnp.exp(tm - m)
        logpt = tm - m - jnp.log(se)
        out_ref[...] = jnp.full_like(out_ref, -jnp.mean(logpt))


def kernel(cos_theta, target):
    b, c = cos_theta.shape
    xt = cos_theta.T
    block_r = min(2000, c)
    grid = pl.cdiv(c, block_r)
    target2 = target.reshape(1, b).astype(jnp.int32)

    out = pl.pallas_call(
        functools.partial(_loss_kernel, c_total=c, block_r=block_r),
        grid=(grid,),
        in_specs=[
            pl.BlockSpec((1, b), lambda k: (0, 0)),
            pl.BlockSpec((block_r, b), lambda k: (k, 0)),
        ],
        out_specs=pl.BlockSpec((1, 1), lambda k: (0, 0)),
        out_shape=jax.ShapeDtypeStruct((1, 1), jnp.float32),
        scratch_shapes=[
            pltpu.VMEM((1, b), jnp.float32),
            pltpu.VMEM((1, b), jnp.float32),
            pltpu.VMEM((1, b), jnp.float32),
        ],
        compiler_params=pltpu.CompilerParams(
            dimension_semantics=("arbitrary",),
        ),
    )(target2, xt)
    return out[0, 0]


# speedup vs baseline: 3.4251x; 1.0130x over previous
import dataclasses
import functools
import math

import jax
import jax.numpy as jnp
from jax import lax
from jax.experimental import pallas as pl
from jax.experimental.pallas import tpu as pltpu
from jax.experimental.pallas import tpu_sc as plsc

_S = 64.0
_M = 0.15
_A = _S * math.log2(math.e)


def _sc_gather(xt, target):
    c, b = xt.shape
    info = plsc.get_sparse_core_info()
    nw = info.num_cores * info.num_subcores
    b_per_w = b // nw
    n_groups = b_per_w // 16
    mesh = plsc.VectorSubcoreMesh(core_axis_name="c", subcore_axis_name="s")
    cp = pltpu.CompilerParams()
    if "needs_layout_passes" in pltpu.CompilerParams.__dataclass_fields__:
        cp = dataclasses.replace(cp, needs_layout_passes=False)

    @functools.partial(
        pl.kernel,
        mesh=mesh,
        compiler_params=cp,
        out_type=jax.ShapeDtypeStruct((b,), jnp.float32),
        scratch_types=[
            pltpu.VMEM((b_per_w,), jnp.int32),
            pltpu.VMEM((b_per_w, b), jnp.float32),
            pltpu.VMEM((b_per_w,), jnp.float32),
        ],
    )
    def gather_kernel(xt_hbm, tgt_hbm, out_hbm, idx_v, rows_v, val_v):
        wid = lax.axis_index("s") * info.num_cores + lax.axis_index("c")
        base = wid * b_per_w
        pltpu.sync_copy(tgt_hbm.at[pl.ds(base, b_per_w)], idx_v)
        pltpu.sync_copy(xt_hbm.at[idx_v], rows_v)
        lane16 = lax.iota(jnp.int32, 16)
        for g in range(n_groups):
            row_idx = lane16 + (g * 16)
            col_idx = lane16 + (base + g * 16)
            vals = plsc.load_gather(rows_v, [row_idx, col_idx])
            val_v[pl.ds(g * 16, 16)] = vals
        pltpu.sync_copy(val_v, out_hbm.at[pl.ds(base, b_per_w)])

    return gather_kernel(xt, target)


def _loss_kernel(tval_ref, x_ref, out_ref, m_sc, s_sc, *, c_total, block_r):
    k = pl.program_id(0)
    nk = pl.num_programs(0)

    @pl.when(k == 0)
    def _init():
        m_sc[...] = jnp.full_like(m_sc, -jnp.inf)
        s_sc[...] = jnp.zeros_like(s_sc)

    def process(x):
        m_old = m_sc[...]
        m_new = jnp.maximum(m_old, jnp.max(x, axis=0, keepdims=True))
        y = jnp.exp2(x * _A - m_new * _A)
        s_sc[...] = s_sc[...] * jnp.exp2(_A * (m_old - m_new)) + jnp.sum(
            y, axis=0, keepdims=True)
        m_sc[...] = m_new

    @pl.when(k < nk - 1)
    def _steady():
        process(x_ref[...])

    @pl.when(k == nk - 1)
    def _last():
        valid = c_total - (nk - 1) * block_r
        x = x_ref[...]
        pad = jax.lax.broadcasted_iota(jnp.int32, x.shape, 0) >= valid
        process(jnp.where(pad, -jnp.inf, x))

        m = _S * m_sc[...]
        tv = _S * tval_ref[...]
        tm = tv - _S * _M
        se = s_sc[...] - jnp.exp(tv - m) + jnp.exp(tm - m)
        logpt = tm - m - jnp.log(se)
        out_ref[...] = jnp.full_like(out_ref, -jnp.mean(logpt))


def kernel(cos_theta, target):
    b, c = cos_theta.shape
    xt = cos_theta.T
    tval = _sc_gather(xt, target.astype(jnp.int32))
    tval2 = tval.reshape(1, b)

    block_r = min(2000, c)
    grid = pl.cdiv(c, block_r)

    out = pl.pallas_call(
        functools.partial(_loss_kernel, c_total=c, block_r=block_r),
        grid=(grid,),
        in_specs=[
            pl.BlockSpec((1, b), lambda k: (0, 0)),
            pl.BlockSpec((block_r, b), lambda k: (k, 0)),
        ],
        out_specs=pl.BlockSpec((1, 1), lambda k: (0, 0)),
        out_shape=jax.ShapeDtypeStruct((1, 1), jnp.float32),
        scratch_shapes=[
            pltpu.VMEM((1, b), jnp.float32),
            pltpu.VMEM((1, b), jnp.float32),
        ],
        compiler_params=pltpu.CompilerParams(
            dimension_semantics=("arbitrary",),
        ),
    )(tval2, xt)
    return out[0, 0]


# speedup vs baseline: 3.6700x; 1.0715x over previous
import dataclasses
import functools
import math

import jax
import jax.numpy as jnp
from jax import lax
from jax.experimental import pallas as pl
from jax.experimental.pallas import tpu as pltpu
from jax.experimental.pallas import tpu_sc as plsc

_S = 64.0
_M = 0.15
_A = _S * math.log2(math.e)


def _sc_gather(xt, target):
    c, b = xt.shape
    info = plsc.get_sparse_core_info()
    nw = info.num_cores * info.num_subcores
    b_per_w = b // nw
    n_groups = b_per_w // 16
    mesh = plsc.VectorSubcoreMesh(core_axis_name="c", subcore_axis_name="s")
    cp = pltpu.CompilerParams()
    if "needs_layout_passes" in pltpu.CompilerParams.__dataclass_fields__:
        cp = dataclasses.replace(cp, needs_layout_passes=False)

    @functools.partial(
        pl.kernel,
        mesh=mesh,
        compiler_params=cp,
        out_type=jax.ShapeDtypeStruct((b,), jnp.float32),
        scratch_types=[
            pltpu.VMEM((b_per_w,), jnp.int32),
            pltpu.VMEM((b_per_w, b), jnp.float32),
            pltpu.VMEM((b_per_w,), jnp.float32),
        ],
    )
    def gather_kernel(xt_hbm, tgt_hbm, out_hbm, idx_v, rows_v, val_v):
        wid = lax.axis_index("s") * info.num_cores + lax.axis_index("c")
        base = wid * b_per_w
        pltpu.sync_copy(tgt_hbm.at[pl.ds(base, b_per_w)], idx_v)
        pltpu.sync_copy(xt_hbm.at[idx_v], rows_v)
        lane16 = lax.iota(jnp.int32, 16)
        for g in range(n_groups):
            row_idx = lane16 + (g * 16)
            col_idx = lane16 + (base + g * 16)
            vals = plsc.load_gather(rows_v, [row_idx, col_idx])
            val_v[pl.ds(g * 16, 16)] = vals
        pltpu.sync_copy(val_v, out_hbm.at[pl.ds(base, b_per_w)])

    return gather_kernel(xt, target)


def _loss_kernel(tval_ref, x_ref, out_ref, m_sc, s_sc, *, c_total, block_r):
    k = pl.program_id(0)
    nk = pl.num_programs(0)

    @pl.when(k == 0)
    def _init():
        m_sc[...] = jnp.full_like(m_sc, -jnp.inf)
        s_sc[...] = jnp.zeros_like(s_sc)

    def process(x):
        m_old = m_sc[...]
        m_new = jnp.maximum(m_old, jnp.max(x, axis=0, keepdims=True))
        y = jnp.exp2(x * _A - m_new * _A)
        s_sc[...] = s_sc[...] * jnp.exp2(_A * (m_old - m_new)) + jnp.sum(
            y, axis=0, keepdims=True)
        m_sc[...] = m_new

    @pl.when(k < nk - 1)
    def _steady():
        process(x_ref[...])

    @pl.when(k == nk - 1)
    def _last():
        valid = c_total - (nk - 1) * block_r
        x = x_ref[...]
        pad = jax.lax.broadcasted_iota(jnp.int32, x.shape, 0) >= valid
        process(jnp.where(pad, -jnp.inf, x))

        m = _S * m_sc[...]
        tv = _S * tval_ref[...]
        tm = tv - _S * _M
        se = s_sc[...] - jnp.exp(tv - m) + jnp.exp(tm - m)
        logpt = tm - m - jnp.log(se)
        out_ref[...] = jnp.full_like(out_ref, -jnp.mean(logpt))


def kernel(cos_theta, target):
    b, c = cos_theta.shape
    xt = cos_theta.T
    tval = _sc_gather(xt, target.astype(jnp.int32))
    tval2 = tval.reshape(1, b)

    block_r = min(4000, c)
    grid = pl.cdiv(c, block_r)

    out = pl.pallas_call(
        functools.partial(_loss_kernel, c_total=c, block_r=block_r),
        grid=(grid,),
        in_specs=[
            pl.BlockSpec((1, b), lambda k: (0, 0)),
            pl.BlockSpec((block_r, b), lambda k: (k, 0)),
        ],
        out_specs=pl.BlockSpec((1, 1), lambda k: (0, 0)),
        out_shape=jax.ShapeDtypeStruct((1, 1), jnp.float32),
        scratch_shapes=[
            pltpu.VMEM((1, b), jnp.float32),
            pltpu.VMEM((1, b), jnp.float32),
        ],
        compiler_params=pltpu.CompilerParams(
            dimension_semantics=("arbitrary",),
        ),
    )(tval2, xt)
    return out[0, 0]


# speedup vs baseline: 3.6966x; 1.0073x over previous
import dataclasses
import functools
import math

import jax
import jax.numpy as jnp
from jax import lax
from jax.experimental import pallas as pl
from jax.experimental.pallas import tpu as pltpu
from jax.experimental.pallas import tpu_sc as plsc

_S = 64.0
_M = 0.15
_A = _S * math.log2(math.e)


def _sc_gather(xt, target):
    c, b = xt.shape
    info = plsc.get_sparse_core_info()
    nw = info.num_cores * info.num_subcores
    b_per_w = b // nw
    n_groups = b_per_w // 16
    mesh = plsc.VectorSubcoreMesh(core_axis_name="c", subcore_axis_name="s")
    cp = pltpu.CompilerParams()
    if "needs_layout_passes" in pltpu.CompilerParams.__dataclass_fields__:
        cp = dataclasses.replace(cp, needs_layout_passes=False)

    @functools.partial(
        pl.kernel,
        mesh=mesh,
        compiler_params=cp,
        out_type=jax.ShapeDtypeStruct((b,), jnp.float32),
        scratch_types=[
            pltpu.VMEM((b_per_w,), jnp.int32),
            pltpu.VMEM((b_per_w, b), jnp.float32),
            pltpu.VMEM((b_per_w,), jnp.float32),
        ],
    )
    def gather_kernel(xt_hbm, tgt_hbm, out_hbm, idx_v, rows_v, val_v):
        wid = lax.axis_index("s") * info.num_cores + lax.axis_index("c")
        base = wid * b_per_w
        pltpu.sync_copy(tgt_hbm.at[pl.ds(base, b_per_w)], idx_v)
        pltpu.sync_copy(xt_hbm.at[idx_v], rows_v)
        lane16 = lax.iota(jnp.int32, 16)
        for g in range(n_groups):
            row_idx = lane16 + (g * 16)
            col_idx = lane16 + (base + g * 16)
            vals = plsc.load_gather(rows_v, [row_idx, col_idx])
            val_v[pl.ds(g * 16, 16)] = vals
        pltpu.sync_copy(val_v, out_hbm.at[pl.ds(base, b_per_w)])

    return gather_kernel(xt, target)


def _loss_kernel(tval_ref, x_ref, out_ref, m_sc, s_sc, *, c_total, block_r):
    k = pl.program_id(0)
    nk = pl.num_programs(0)

    @pl.when(k == 0)
    def _init():
        m_sc[...] = jnp.full_like(m_sc, -jnp.inf)
        s_sc[...] = jnp.zeros_like(s_sc)

    def process(x):
        m_old = m_sc[...]
        m_new = jnp.maximum(m_old, jnp.max(x, axis=0, keepdims=True))
        y = jnp.exp2(x * _A - m_new * _A)
        s_sc[...] = s_sc[...] * jnp.exp2(_A * (m_old - m_new)) + jnp.sum(
            y, axis=0, keepdims=True)
        m_sc[...] = m_new

    @pl.when(k < nk - 1)
    def _steady():
        process(x_ref[...])

    @pl.when(k == nk - 1)
    def _last():
        valid = c_total - (nk - 1) * block_r
        x = x_ref[...]
        pad = jax.lax.broadcasted_iota(jnp.int32, x.shape, 0) >= valid
        process(jnp.where(pad, -jnp.inf, x))

        m = _S * m_sc[...]
        tv = _S * tval_ref[...]
        tm = tv - _S * _M
        se = s_sc[...] - jnp.exp(tv - m) + jnp.exp(tm - m)
        logpt = tm - m - jnp.log(se)
        out_ref[...] = jnp.full_like(out_ref, -jnp.mean(logpt))


def kernel(cos_theta, target):
    b, c = cos_theta.shape
    xt = cos_theta.T
    tval = _sc_gather(xt, target.astype(jnp.int32))
    tval2 = tval.reshape(1, b)

    block_r = min(5000, c)
    grid = pl.cdiv(c, block_r)

    out = pl.pallas_call(
        functools.partial(_loss_kernel, c_total=c, block_r=block_r),
        grid=(grid,),
        in_specs=[
            pl.BlockSpec((1, b), lambda k: (0, 0)),
            pl.BlockSpec((block_r, b), lambda k: (k, 0)),
        ],
        out_specs=pl.BlockSpec((1, 1), lambda k: (0, 0)),
        out_shape=jax.ShapeDtypeStruct((1, 1), jnp.float32),
        scratch_shapes=[
            pltpu.VMEM((1, b), jnp.float32),
            pltpu.VMEM((1, b), jnp.float32),
        ],
        compiler_params=pltpu.CompilerParams(
            dimension_semantics=("arbitrary",),
        ),
    )(tval2, xt)
    return out[0, 0]


# speedup vs baseline: 3.7981x; 1.0274x over previous
import dataclasses
import functools
import math

import jax
import jax.numpy as jnp
from jax import lax
from jax.experimental import pallas as pl
from jax.experimental.pallas import tpu as pltpu
from jax.experimental.pallas import tpu_sc as plsc

_S = 64.0
_M = 0.15
_A = _S * math.log2(math.e)


def _sc_gather(xt, target):
    c, b = xt.shape
    info = plsc.get_sparse_core_info()
    nw = info.num_cores * info.num_subcores
    b_per_w = b // nw
    n_groups = b_per_w // 16
    mesh = plsc.VectorSubcoreMesh(core_axis_name="c", subcore_axis_name="s")
    cp = pltpu.CompilerParams()
    if "needs_layout_passes" in pltpu.CompilerParams.__dataclass_fields__:
        cp = dataclasses.replace(cp, needs_layout_passes=False)

    @functools.partial(
        pl.kernel,
        mesh=mesh,
        compiler_params=cp,
        out_type=jax.ShapeDtypeStruct((b,), jnp.float32),
        scratch_types=[
            pltpu.VMEM((b_per_w,), jnp.int32),
            pltpu.VMEM((b_per_w, b), jnp.float32),
            pltpu.VMEM((b_per_w,), jnp.float32),
        ],
    )
    def gather_kernel(xt_hbm, tgt_hbm, out_hbm, idx_v, rows_v, val_v):
        wid = lax.axis_index("s") * info.num_cores + lax.axis_index("c")
        base = wid * b_per_w
        pltpu.sync_copy(tgt_hbm.at[pl.ds(base, b_per_w)], idx_v)
        pltpu.sync_copy(xt_hbm.at[idx_v], rows_v)
        lane16 = lax.iota(jnp.int32, 16)
        for g in range(n_groups):
            row_idx = lane16 + (g * 16)
            col_idx = lane16 + (base + g * 16)
            vals = plsc.load_gather(rows_v, [row_idx, col_idx])
            val_v[pl.ds(g * 16, 16)] = vals
        pltpu.sync_copy(val_v, out_hbm.at[pl.ds(base, b_per_w)])

    return gather_kernel(xt, target)


def _stream_kernel(x_ref, m_ref, s_ref, m_sc, s_sc, *, c_total, block_r):
    k = pl.program_id(0)
    nk = pl.num_programs(0)

    @pl.when(k == 0)
    def _init():
        m_sc[...] = jnp.full_like(m_sc, -jnp.inf)
        s_sc[...] = jnp.zeros_like(s_sc)

    def process(x):
        m_old = m_sc[...]
        m_new = jnp.maximum(m_old, jnp.max(x, axis=0, keepdims=True))
        y = jnp.exp2(x * _A - m_new * _A)
        s_sc[...] = s_sc[...] * jnp.exp2(_A * (m_old - m_new)) + jnp.sum(
            y, axis=0, keepdims=True)
        m_sc[...] = m_new

    @pl.when(k < nk - 1)
    def _steady():
        process(x_ref[...])

    @pl.when(k == nk - 1)
    def _last():
        valid = c_total - (nk - 1) * block_r
        x = x_ref[...]
        pad = jax.lax.broadcasted_iota(jnp.int32, x.shape, 0) >= valid
        process(jnp.where(pad, -jnp.inf, x))
        m_ref[...] = m_sc[...]
        s_ref[...] = s_sc[...]


def _epilogue_kernel(tval_ref, m_ref, s_ref, out_ref):
    m = _S * m_ref[...]
    tv = _S * tval_ref[...]
    tm = tv - _S * _M
    se = s_ref[...] - jnp.exp(tv - m) + jnp.exp(tm - m)
    logpt = tm - m - jnp.log(se)
    out_ref[...] = jnp.full_like(out_ref, -jnp.mean(logpt))


def kernel(cos_theta, target):
    b, c = cos_theta.shape
    xt = cos_theta.T
    tval = _sc_gather(xt, target.astype(jnp.int32))

    block_r = min(5000, c)
    grid = pl.cdiv(c, block_r)

    m_v, s_v = pl.pallas_call(
        functools.partial(_stream_kernel, c_total=c, block_r=block_r),
        grid=(grid,),
        in_specs=[pl.BlockSpec((block_r, b), lambda k: (k, 0))],
        out_specs=[
            pl.BlockSpec((1, b), lambda k: (0, 0)),
            pl.BlockSpec((1, b), lambda k: (0, 0)),
        ],
        out_shape=[
            jax.ShapeDtypeStruct((1, b), jnp.float32),
            jax.ShapeDtypeStruct((1, b), jnp.float32),
        ],
        scratch_shapes=[
            pltpu.VMEM((1, b), jnp.float32),
            pltpu.VMEM((1, b), jnp.float32),
        ],
        compiler_params=pltpu.CompilerParams(
            dimension_semantics=("arbitrary",),
        ),
    )(xt)

    out = pl.pallas_call(
        _epilogue_kernel,
        in_specs=[
            pl.BlockSpec((1, b), lambda: (0, 0)),
            pl.BlockSpec((1, b), lambda: (0, 0)),
            pl.BlockSpec((1, b), lambda: (0, 0)),
        ],
        out_specs=pl.BlockSpec((1, 1), lambda: (0, 0)),
        out_shape=jax.ShapeDtypeStruct((1, 1), jnp.float32),
    )(tval.reshape(1, b), m_v, s_v)
    return out[0, 0]


# speedup vs baseline: 3.8035x; 1.0014x over previous
import dataclasses
import functools
import math

import jax
import jax.numpy as jnp
from jax import lax
from jax.experimental import pallas as pl
from jax.experimental.pallas import tpu as pltpu
from jax.experimental.pallas import tpu_sc as plsc

_S = 64.0
_M = 0.15
_A = _S * math.log2(math.e)


def _sc_gather(xt, target):
    c, b = xt.shape
    info = plsc.get_sparse_core_info()
    nw = info.num_cores * info.num_subcores
    b_per_w = b // nw
    n_groups = b_per_w // 16
    mesh = plsc.VectorSubcoreMesh(core_axis_name="c", subcore_axis_name="s")
    cp = pltpu.CompilerParams()
    if "needs_layout_passes" in pltpu.CompilerParams.__dataclass_fields__:
        cp = dataclasses.replace(cp, needs_layout_passes=False)

    @functools.partial(
        pl.kernel,
        mesh=mesh,
        compiler_params=cp,
        out_type=jax.ShapeDtypeStruct((b,), jnp.float32),
        scratch_types=[
            pltpu.VMEM((b_per_w,), jnp.int32),
            pltpu.VMEM((b_per_w, b), jnp.float32),
            pltpu.VMEM((b_per_w,), jnp.float32),
        ],
    )
    def gather_kernel(xt_hbm, tgt_hbm, out_hbm, idx_v, rows_v, val_v):
        wid = lax.axis_index("s") * info.num_cores + lax.axis_index("c")
        base = wid * b_per_w
        pltpu.sync_copy(tgt_hbm.at[pl.ds(base, b_per_w)], idx_v)
        pltpu.sync_copy(xt_hbm.at[idx_v], rows_v)
        lane16 = lax.iota(jnp.int32, 16)
        for g in range(n_groups):
            row_idx = lane16 + (g * 16)
            col_idx = lane16 + (base + g * 16)
            vals = plsc.load_gather(rows_v, [row_idx, col_idx])
            val_v[pl.ds(g * 16, 16)] = vals
        pltpu.sync_copy(val_v, out_hbm.at[pl.ds(base, b_per_w)])

    return gather_kernel(xt, target)


def _stream_kernel(x_ref, m_ref, s_ref, m_sc, s_sc, *, c_total, block_r):
    k = pl.program_id(0)
    nk = pl.num_programs(0)

    @pl.when(k == 0)
    def _init():
        m_sc[...] = jnp.full_like(m_sc, -jnp.inf)
        s_sc[...] = jnp.zeros_like(s_sc)

    def process(x):
        m_old = m_sc[...]
        m_new = jnp.maximum(m_old, jnp.max(x, axis=0, keepdims=True))
        y = jnp.exp2(x * _A - m_new * _A)
        s_sc[...] = s_sc[...] * jnp.exp2(_A * (m_old - m_new)) + jnp.sum(
            y, axis=0, keepdims=True)
        m_sc[...] = m_new

    valid = c_total - (nk - 1) * block_r
    if valid == block_r:
        process(x_ref[...])
    else:
        @pl.when(k < nk - 1)
        def _steady():
            process(x_ref[...])

        @pl.when(k == nk - 1)
        def _last():
            x = x_ref[...]
            pad = jax.lax.broadcasted_iota(jnp.int32, x.shape, 0) >= valid
            process(jnp.where(pad, -jnp.inf, x))

    @pl.when(k == nk - 1)
    def _emit():
        m_ref[...] = m_sc[...]
        s_ref[...] = s_sc[...]


def _epilogue_kernel(tval_ref, m_ref, s_ref, out_ref):
    m = _S * m_ref[...]
    tv = _S * tval_ref[...]
    tm = tv - _S * _M
    se = s_ref[...] - jnp.exp(tv - m) + jnp.exp(tm - m)
    logpt = tm - m - jnp.log(se)
    out_ref[...] = jnp.full_like(out_ref, -jnp.mean(logpt))


def kernel(cos_theta, target):
    b, c = cos_theta.shape
    xt = cos_theta.T
    tval = _sc_gather(xt, target.astype(jnp.int32))

    block_r = min(5000, c)
    grid = pl.cdiv(c, block_r)

    m_v, s_v = pl.pallas_call(
        functools.partial(_stream_kernel, c_total=c, block_r=block_r),
        grid=(grid,),
        in_specs=[pl.BlockSpec((block_r, b), lambda k: (k, 0))],
        out_specs=[
            pl.BlockSpec((1, b), lambda k: (0, 0)),
            pl.BlockSpec((1, b), lambda k: (0, 0)),
        ],
        out_shape=[
            jax.ShapeDtypeStruct((1, b), jnp.float32),
            jax.ShapeDtypeStruct((1, b), jnp.float32),
        ],
        scratch_shapes=[
            pltpu.VMEM((1, b), jnp.float32),
            pltpu.VMEM((1, b), jnp.float32),
        ],
        compiler_params=pltpu.CompilerParams(
            dimension_semantics=("arbitrary",),
        ),
    )(xt)

    out = pl.pallas_call(
        _epilogue_kernel,
        in_specs=[
            pl.BlockSpec((1, b), lambda: (0, 0)),
            pl.BlockSpec((1, b), lambda: (0, 0)),
            pl.BlockSpec((1, b), lambda: (0, 0)),
        ],
        out_specs=pl.BlockSpec((1, 1), lambda: (0, 0)),
        out_shape=jax.ShapeDtypeStruct((1, 1), jnp.float32),
    )(tval.reshape(1, b), m_v, s_v)
    return out[0, 0]


# speedup vs baseline: 3.8668x; 1.0166x over previous
import dataclasses
import functools
import math

import jax
import jax.numpy as jnp
from jax import lax
from jax.experimental import pallas as pl
from jax.experimental.pallas import tpu as pltpu
from jax.experimental.pallas import tpu_sc as plsc

_S = 64.0
_M = 0.15
_A = _S * math.log2(math.e)


def _sc_gather(xt, target):
    c, b = xt.shape
    info = plsc.get_sparse_core_info()
    nw = info.num_cores * info.num_subcores
    b_per_w = b // nw
    n_groups = b_per_w // 16
    mesh = plsc.VectorSubcoreMesh(core_axis_name="c", subcore_axis_name="s")
    cp = pltpu.CompilerParams()
    if "needs_layout_passes" in pltpu.CompilerParams.__dataclass_fields__:
        cp = dataclasses.replace(cp, needs_layout_passes=False)

    @functools.partial(
        pl.kernel,
        mesh=mesh,
        compiler_params=cp,
        out_type=jax.ShapeDtypeStruct((b,), jnp.float32),
        scratch_types=[
            pltpu.VMEM((b_per_w,), jnp.int32),
            pltpu.VMEM((b_per_w, b), jnp.float32),
            pltpu.VMEM((b_per_w,), jnp.float32),
        ],
    )
    def gather_kernel(xt_hbm, tgt_hbm, out_hbm, idx_v, rows_v, val_v):
        wid = lax.axis_index("s") * info.num_cores + lax.axis_index("c")
        base = wid * b_per_w
        pltpu.sync_copy(tgt_hbm.at[pl.ds(base, b_per_w)], idx_v)
        pltpu.sync_copy(xt_hbm.at[idx_v], rows_v)
        lane16 = lax.iota(jnp.int32, 16)
        for g in range(n_groups):
            row_idx = lane16 + (g * 16)
            col_idx = lane16 + (base + g * 16)
            vals = plsc.load_gather(rows_v, [row_idx, col_idx])
            val_v[pl.ds(g * 16, 16)] = vals
        pltpu.sync_copy(val_v, out_hbm.at[pl.ds(base, b_per_w)])

    return gather_kernel(xt, target)


def _stream_kernel(x0_ref, x1_ref, m_ref, s_ref, m_sc, s_sc, *, c_total,
                   block_r):
    k = pl.program_id(0)
    nk = pl.num_programs(0)

    @pl.when(k == 0)
    def _init():
        m_sc[...] = jnp.full_like(m_sc, -jnp.inf)
        s_sc[...] = jnp.zeros_like(s_sc)

    def process(x):
        m_old = m_sc[...]
        m_new = jnp.maximum(m_old, jnp.max(x, axis=0, keepdims=True))
        y = jnp.exp2(x * _A - m_new * _A)
        s_sc[...] = s_sc[...] * jnp.exp2(_A * (m_old - m_new)) + jnp.sum(
            y, axis=0, keepdims=True)
        m_sc[...] = m_new

    if c_total == 2 * nk * block_r:
        process(x0_ref[...])
        process(x1_ref[...])
    else:
        @pl.when(k < nk - 1)
        def _steady():
            process(x0_ref[...])
            process(x1_ref[...])

        @pl.when(k == nk - 1)
        def _last():
            valid = c_total - 2 * (nk - 1) * block_r
            for j, x_ref in enumerate((x0_ref, x1_ref)):
                x = x_ref[...]
                pad = (jax.lax.broadcasted_iota(jnp.int32, x.shape, 0)
                       >= valid - j * block_r)
                process(jnp.where(pad, -jnp.inf, x))

    @pl.when(k == nk - 1)
    def _emit():
        m_ref[...] = m_sc[...]
        s_ref[...] = s_sc[...]


def _epilogue_kernel(tval_ref, m_ref, s_ref, out_ref):
    m = _S * m_ref[...]
    tv = _S * tval_ref[...]
    tm = tv - _S * _M
    se = s_ref[...] - jnp.exp(tv - m) + jnp.exp(tm - m)
    logpt = tm - m - jnp.log(se)
    out_ref[...] = jnp.full_like(out_ref, -jnp.mean(logpt))


def kernel(cos_theta, target):
    b, c = cos_theta.shape
    xt = cos_theta.T
    tval = _sc_gather(xt, target.astype(jnp.int32))

    block_r = min(2000, c)
    grid = pl.cdiv(c, 2 * block_r)

    m_v, s_v = pl.pallas_call(
        functools.partial(_stream_kernel, c_total=c, block_r=block_r),
        grid=(grid,),
        in_specs=[
            pl.BlockSpec((block_r, b), lambda k: (2 * k, 0)),
            pl.BlockSpec((block_r, b), lambda k: (2 * k + 1, 0)),
        ],
        out_specs=[
            pl.BlockSpec((1, b), lambda k: (0, 0)),
            pl.BlockSpec((1, b), lambda k: (0, 0)),
        ],
        out_shape=[
            jax.ShapeDtypeStruct((1, b), jnp.float32),
            jax.ShapeDtypeStruct((1, b), jnp.float32),
        ],
        scratch_shapes=[
            pltpu.VMEM((1, b), jnp.float32),
            pltpu.VMEM((1, b), jnp.float32),
        ],
        compiler_params=pltpu.CompilerParams(
            dimension_semantics=("arbitrary",),
        ),
    )(xt, xt)

    out = pl.pallas_call(
        _epilogue_kernel,
        in_specs=[
            pl.BlockSpec((1, b), lambda: (0, 0)),
            pl.BlockSpec((1, b), lambda: (0, 0)),
            pl.BlockSpec((1, b), lambda: (0, 0)),
        ],
        out_specs=pl.BlockSpec((1, 1), lambda: (0, 0)),
        out_shape=jax.ShapeDtypeStruct((1, 1), jnp.float32),
    )(tval.reshape(1, b), m_v, s_v)
    return out[0, 0]
